# baseline (speedup 1.0000x reference)
import functools

import jax
import jax.numpy as jnp
from jax import lax
from jax.experimental import pallas as pl
from jax.experimental.pallas import tpu as pltpu
from jax.experimental.pallas import tpu_sc as plsc

BATCH = 16384
EMBED = 64
NC = 2
NS = 16
NW = NC * NS

GCOLS = 256
SHIFT = 8
NBUF = 4
OUT_ROWS = 17408
ROWS_BLK = 1024
N_BLK = BATCH // ROWS_BLK
STRIP = BATCH // 16

I32 = jnp.int32


def _gather_call(n_rows):
    n_groups = n_rows // GCOLS + 1
    nb_base = n_groups // NW
    nb_rem = n_groups % NW

    def body(ids_hbm, tab_hbm, tail_hbm, out_hbm,
             idsv, hpos, spos, counts, offs, cursor, slabbuf, outbuf,
             sem, sem2):
        w = lax.axis_index("s") * NC + lax.axis_index("c")
        lo = w * nb_base + jnp.minimum(w, nb_rem)
        nb = nb_base + jnp.where(w < nb_rem, 1, 0)
        iota = lax.iota(I32, 16)
        ones = jnp.ones((16,), I32)
        zeros = jnp.zeros((16,), I32)
        strip_base = iota * STRIP

        def fire(g_rel, parity):
            g = lo + g_rel
            is_tail = g == n_groups - 1

            @pl.when(jnp.logical_not(is_tail))
            def _():
                pltpu.async_copy(tab_hbm.at[:, pl.ds(g * GCOLS, GCOLS)],
                                 slabbuf.at[parity], sem)

            @pl.when(is_tail)
            def _():
                pltpu.async_copy(tail_hbm, slabbuf.at[parity], sem)

        fire(0, 0)
        for i in range(1, NBUF):
            @pl.when(nb > i)
            def _(i=i):
                fire(i, i)

        pltpu.sync_copy(ids_hbm, idsv)
        for c in range(9):
            counts[pl.ds(c * 16, 16)] = zeros

        def p1(k, cur):
            u = plsc.load_gather(idsv, [strip_base + k])
            bg = u >> SHIFT
            m = (bg >= lo) & (bg < lo + nb)
            bl = jnp.where(m, bg - lo, 0)
            plsc.addupdate_scatter(counts, [bl], ones, mask=m)
            plsc.store_scatter(hpos, [strip_base + cur], strip_base + k,
                               mask=m)
            return cur + jnp.where(m, 1, 0)

        cur16 = lax.fori_loop(0, STRIP, p1, jnp.zeros((16,), I32))

        car = jnp.asarray(0, I32)
        for c in range(9):
            v = counts[pl.ds(c * 16, 16)]
            s = plsc.cumsum(v)
            e = s - v + car
            offs[pl.ds(c * 16, 16)] = e
            cursor[pl.ds(c * 16, 16)] = e
            car = car + s[15]

        def strip(j, carry):
            cnt = cur16[j]

            def p3(k, carry2):
                base = j * STRIP + k * 16
                valid = (k * 16 + iota) < cnt
                p = hpos[pl.ds(base, 16)] & (BATCH - 1)
                u = plsc.load_gather(idsv, [p])
                bl = jnp.where(valid, (u >> SHIFT) - lo, 0)
                boff = plsc.load_gather(cursor, [bl])
                dup, lastm = plsc.scan_count(bl, valid)
                plsc.store_scatter(spos, [boff + dup - 1], p, mask=valid)
                plsc.addupdate_scatter(cursor, [bl], dup,
                                       mask=lastm & valid)
                return carry2

            lax.fori_loop(0, (cnt + 15) >> 4, p3, jnp.asarray(0, I32))
            return carry

        for j in range(16):
            strip(j, 0)

        def group_step(g_rel, wcnt):
            parity = g_rel & (NBUF - 1)
            pltpu.make_async_copy(tail_hbm, slabbuf.at[parity], sem).wait()

            ov = plsc.load_gather(offs, [jnp.minimum(g_rel + iota, 143)])
            st, en = ov[0], ov[1]
            pb = jnp.full((16,), parity, I32)

            def window(k, wc):
                base = st + k * 16
                valid = (base + iota) < en
                hp = spos[pl.ds(base, 16)] & (BATCH - 1)
                hu = plsc.load_gather(idsv, [hp])
                lane = hu & (GCOLS - 1)

                @pl.when(wc >= 4)
                def _():
                    pltpu.make_async_copy(
                        outbuf.at[pl.ds(0, 16), :],
                        out_hbm.at[16384 + iota], sem2).wait()

                rowv = (wc & 3) * 16 + iota
                for d in range(EMBED):
                    dsp = jnp.full((16,), d, I32)
                    val = plsc.load_gather(slabbuf, [pb, dsp, lane])
                    plsc.store_scatter(outbuf, [rowv, dsp], val)
                rows_dst = jnp.where(valid, hp, 16384 + iota)
                pltpu.async_copy(outbuf.at[pl.ds((wc & 3) * 16, 16), :],
                                 out_hbm.at[rows_dst], sem2)
                return wc + 1

            nwin = (en - st + 15) >> 4
            wcnt = lax.fori_loop(0, nwin, window, wcnt)

            @pl.when(g_rel + NBUF < nb)
            def _():
                fire(g_rel + NBUF, parity)

            return wcnt

        wcnt = lax.fori_loop(0, nb, group_step, jnp.asarray(0, I32))

        def drain(_, c):
            pltpu.make_async_copy(outbuf.at[pl.ds(0, 16), :],
                                  out_hbm.at[16384 + iota], sem2).wait()
            return c

        lax.fori_loop(0, jnp.minimum(wcnt, 4), drain, jnp.asarray(0, I32))

    mesh = plsc.VectorSubcoreMesh(core_axis_name="c", subcore_axis_name="s",
                                  num_cores=NC, num_subcores=NS)
    return pl.kernel(
        body,
        out_type=jax.ShapeDtypeStruct((OUT_ROWS, 128), jnp.float32),
        mesh=mesh,
        compiler_params=pltpu.CompilerParams(needs_layout_passes=False,
                                             use_tc_tiling_on_sc=True),
        scratch_types=[
            pltpu.VMEM((BATCH,), I32),
            pltpu.VMEM((BATCH,), I32),
            pltpu.VMEM((BATCH + 16,), I32),
            pltpu.VMEM((144,), I32),
            pltpu.VMEM((144,), I32),
            pltpu.VMEM((144,), I32),
            pltpu.VMEM((NBUF, EMBED, GCOLS), jnp.float32),
            pltpu.VMEM((64, 128), jnp.float32),
            pltpu.SemaphoreType.DMA,
            pltpu.SemaphoreType.DMA,
        ],
    )


def _combine_body(u_ref, p_ref, wb_ref, out_ref):
    wrow = wb_ref[0, :EMBED]
    bias = wb_ref[0, EMBED]
    prod = u_ref[0][:, :EMBED] * p_ref[0][:, :EMBED] * wrow[None, :]
    out_ref[0, 0, :] = jnp.sum(prod, axis=1) + bias


def _tc_combine(u_rows, p_rows, wb):
    u3 = u_rows.reshape(OUT_ROWS // ROWS_BLK, ROWS_BLK, 128)
    p3 = p_rows.reshape(OUT_ROWS // ROWS_BLK, ROWS_BLK, 128)
    out = pl.pallas_call(
        _combine_body,
        grid=(N_BLK,),
        in_specs=[
            pl.BlockSpec((1, ROWS_BLK, 128), lambda i: (i, 0, 0)),
            pl.BlockSpec((1, ROWS_BLK, 128), lambda i: (i, 0, 0)),
            pl.BlockSpec((1, EMBED + 16), lambda i: (0, 0)),
        ],
        out_specs=pl.BlockSpec((1, 1, ROWS_BLK), lambda i: (i, 0, 0)),
        out_shape=jax.ShapeDtypeStruct((N_BLK, 1, ROWS_BLK), jnp.float32),
    )(u3, p3, wb)
    return out.reshape(BATCH)


@jax.jit
def _run(uid, pid, utab_t, ptab_t, utail, ptail, wb):
    u_rows = _gather_call(1000000)(uid, utab_t, utail)
    p_rows = _gather_call(100000)(pid, ptab_t, ptail)
    return _tc_combine(u_rows, p_rows, wb)


def kernel(user_ids, product_ids, user_embedding, product_embedding, fc_w, fc_b):
    uid = user_ids.astype(I32)
    pid = product_ids.astype(I32)
    n_u, n_p = user_embedding.shape[0], product_embedding.shape[0]
    utail = jnp.pad(user_embedding[n_u - n_u % GCOLS:].T,
                    ((0, 0), (0, GCOLS - n_u % GCOLS)))
    ptail = jnp.pad(product_embedding[n_p - n_p % GCOLS:].T,
                    ((0, 0), (0, GCOLS - n_p % GCOLS)))
    wb = jnp.concatenate(
        [fc_w.reshape(EMBED), jnp.broadcast_to(fc_b.reshape(1), (16,))])
    return _run(uid, pid, user_embedding.T, product_embedding.T,
                utail, ptail, wb.reshape(1, EMBED + 16))


# speedup vs baseline: 1.4174x; 1.4174x over previous
import functools

import jax
import jax.numpy as jnp
from jax import lax
from jax.experimental import pallas as pl
from jax.experimental.pallas import tpu as pltpu
from jax.experimental.pallas import tpu_sc as plsc

BATCH = 16384
EMBED = 64
NC = 2
NS = 16
NW = NC * NS

GCOLS = 512
SHIFT = 9
OUT_ROWS = 17408
ROWS_BLK = 1024
N_BLK = BATCH // ROWS_BLK
STRIP = BATCH // 16

I32 = jnp.int32


def _fused_body(uid_hbm, pid_hbm, utab_hbm, ptab_hbm, utail_hbm, ptail_hbm,
                uout_hbm, pout_hbm,
                idsv, hpos, spos, counts, offs, cursor, slabbuf, outbuf,
                sem, sem2):
    w = lax.axis_index("s") * NC + lax.axis_index("c")
    iota = lax.iota(I32, 16)
    ones = jnp.ones((16,), I32)
    zeros = jnp.zeros((16,), I32)
    strip_base = iota * STRIP

    def run_table(ids_hbm, tab_hbm, tail_hbm, out_hbm, n_rows):
        n_groups = n_rows // GCOLS + 1
        nb_base = n_groups // NW
        nb_rem = n_groups % NW
        lo = w * nb_base + jnp.minimum(w, nb_rem)
        nb = nb_base + jnp.where(w < nb_rem, 1, 0)

        def fire(g_rel, parity):
            g = lo + g_rel
            is_tail = g == n_groups - 1

            @pl.when(jnp.logical_not(is_tail))
            def _():
                pltpu.async_copy(tab_hbm.at[:, pl.ds(g * GCOLS, GCOLS)],
                                 slabbuf.at[parity], sem)

            @pl.when(is_tail)
            def _():
                pltpu.async_copy(tail_hbm, slabbuf.at[parity], sem)

        fire(0, 0)

        @pl.when(nb > 1)
        def _():
            fire(1, 1)

        pltpu.sync_copy(ids_hbm, idsv)
        for c in range(9):
            counts[pl.ds(c * 16, 16)] = zeros

        def p1(k4, cur):
            for u4 in range(4):
                k = k4 * 4 + u4
                u = plsc.load_gather(idsv, [strip_base + k])
                bg = u >> SHIFT
                m = (bg >= lo) & (bg < lo + nb)
                bl = jnp.where(m, bg - lo, 0)
                plsc.addupdate_scatter(counts, [bl], ones, mask=m)
                plsc.store_scatter(hpos, [strip_base + cur],
                                   strip_base + k, mask=m)
                cur = cur + jnp.where(m, 1, 0)
            return cur

        cur16 = lax.fori_loop(0, STRIP // 4, p1, jnp.zeros((16,), I32))

        car = jnp.asarray(0, I32)
        for c in range(9):
            v = counts[pl.ds(c * 16, 16)]
            s = plsc.cumsum(v)
            e = s - v + car
            offs[pl.ds(c * 16, 16)] = e
            cursor[pl.ds(c * 16, 16)] = e
            car = car + s[15]

        for j in range(16):
            cnt = cur16[j]

            def p3(k, carry2, j=j, cnt=cnt):
                base = j * STRIP + k * 16
                valid = (k * 16 + iota) < cnt
                p = hpos[pl.ds(base, 16)] & (BATCH - 1)
                u = plsc.load_gather(idsv, [p])
                bl = jnp.where(valid, (u >> SHIFT) - lo, 0)
                boff = plsc.load_gather(cursor, [bl])
                dup, lastm = plsc.scan_count(bl, valid)
                plsc.store_scatter(spos, [boff + dup - 1], p, mask=valid)
                plsc.addupdate_scatter(cursor, [bl], dup,
                                       mask=lastm & valid)
                return carry2

            lax.fori_loop(0, (cnt + 15) >> 4, p3, jnp.asarray(0, I32))

        def group_step(g_rel, wcnt):
            parity = g_rel & 1
            pltpu.make_async_copy(tail_hbm, slabbuf.at[parity], sem).wait()

            ov = plsc.load_gather(offs, [jnp.minimum(g_rel + iota, 143)])
            st, en = ov[0], ov[1]
            pb = jnp.full((16,), parity, I32)

            def window(k, wc):
                base = st + k * 16
                valid = (base + iota) < en
                hp = spos[pl.ds(base, 16)] & (BATCH - 1)
                hu = plsc.load_gather(idsv, [hp])
                lane = hu & (GCOLS - 1)

                @pl.when(wc >= 4)
                def _():
                    pltpu.make_async_copy(
                        outbuf.at[pl.ds(0, 16), :],
                        out_hbm.at[16384 + iota], sem2).wait()

                rowv = (wc & 3) * 16 + iota
                for d in range(EMBED):
                    dsp = jnp.full((16,), d, I32)
                    val = plsc.load_gather(slabbuf, [pb, dsp, lane])
                    plsc.store_scatter(outbuf, [rowv, dsp], val)
                rows_dst = jnp.where(valid, hp, 16384 + iota)
                pltpu.async_copy(outbuf.at[pl.ds((wc & 3) * 16, 16), :],
                                 out_hbm.at[rows_dst], sem2)
                return wc + 1

            nwin = (en - st + 15) >> 4
            wcnt = lax.fori_loop(0, nwin, window, wcnt)

            @pl.when(g_rel + 2 < nb)
            def _():
                fire(g_rel + 2, parity)

            return wcnt

        wcnt = lax.fori_loop(0, nb, group_step, jnp.asarray(0, I32))

        def drain(_, c):
            pltpu.make_async_copy(outbuf.at[pl.ds(0, 16), :],
                                  out_hbm.at[16384 + iota], sem2).wait()
            return c

        lax.fori_loop(0, jnp.minimum(wcnt, 4), drain, jnp.asarray(0, I32))

    run_table(pid_hbm, ptab_hbm, ptail_hbm, pout_hbm, 100000)
    run_table(uid_hbm, utab_hbm, utail_hbm, uout_hbm, 1000000)


def _fused_call():
    mesh = plsc.VectorSubcoreMesh(core_axis_name="c", subcore_axis_name="s",
                                  num_cores=NC, num_subcores=NS)
    return pl.kernel(
        _fused_body,
        out_type=(jax.ShapeDtypeStruct((OUT_ROWS, 128), jnp.float32),
                  jax.ShapeDtypeStruct((OUT_ROWS, 128), jnp.float32)),
        mesh=mesh,
        compiler_params=pltpu.CompilerParams(needs_layout_passes=False,
                                             use_tc_tiling_on_sc=True),
        scratch_types=[
            pltpu.VMEM((BATCH,), I32),
            pltpu.VMEM((BATCH,), I32),
            pltpu.VMEM((BATCH + 16,), I32),
            pltpu.VMEM((144,), I32),
            pltpu.VMEM((144,), I32),
            pltpu.VMEM((144,), I32),
            pltpu.VMEM((2, EMBED, GCOLS), jnp.float32),
            pltpu.VMEM((64, 128), jnp.float32),
            pltpu.SemaphoreType.DMA,
            pltpu.SemaphoreType.DMA,
        ],
    )


def _combine_body(u_ref, p_ref, wb_ref, out_ref):
    wrow = wb_ref[0, :EMBED]
    bias = wb_ref[0, EMBED]
    prod = u_ref[0][:, :EMBED] * p_ref[0][:, :EMBED] * wrow[None, :]
    out_ref[0, 0, :] = jnp.sum(prod, axis=1) + bias


def _tc_combine(u_rows, p_rows, wb):
    u3 = u_rows.reshape(OUT_ROWS // ROWS_BLK, ROWS_BLK, 128)
    p3 = p_rows.reshape(OUT_ROWS // ROWS_BLK, ROWS_BLK, 128)
    out = pl.pallas_call(
        _combine_body,
        grid=(N_BLK,),
        in_specs=[
            pl.BlockSpec((1, ROWS_BLK, 128), lambda i: (i, 0, 0)),
            pl.BlockSpec((1, ROWS_BLK, 128), lambda i: (i, 0, 0)),
            pl.BlockSpec((1, EMBED + 16), lambda i: (0, 0)),
        ],
        out_specs=pl.BlockSpec((1, 1, ROWS_BLK), lambda i: (i, 0, 0)),
        out_shape=jax.ShapeDtypeStruct((N_BLK, 1, ROWS_BLK), jnp.float32),
    )(u3, p3, wb)
    return out.reshape(BATCH)


@jax.jit
def _run(uid, pid, utab_t, ptab_t, utail, ptail, wb):
    u_rows, p_rows = _fused_call()(uid, pid, utab_t, ptab_t, utail, ptail)
    return _tc_combine(u_rows, p_rows, wb)


def kernel(user_ids, product_ids, user_embedding, product_embedding, fc_w, fc_b):
    uid = user_ids.astype(I32)
    pid = product_ids.astype(I32)
    n_u, n_p = user_embedding.shape[0], product_embedding.shape[0]
    utail = jnp.pad(user_embedding[n_u - n_u % GCOLS:].T,
                    ((0, 0), (0, GCOLS - n_u % GCOLS)))
    ptail = jnp.pad(product_embedding[n_p - n_p % GCOLS:].T,
                    ((0, 0), (0, GCOLS - n_p % GCOLS)))
    wb = jnp.concatenate(
        [fc_w.reshape(EMBED), jnp.broadcast_to(fc_b.reshape(1), (16,))])
    return _run(uid, pid, user_embedding.T, product_embedding.T,
                utail, ptail, wb.reshape(1, EMBED + 16))
